# baseline (speedup 1.0000x reference)
import jax
import jax.numpy as jnp
from jax.experimental import pallas as pl
from jax.experimental.pallas import tpu as pltpu

N, C, T = 64, 256, 512
K = 8
ALPHA = 0.8
B = 16


def _dyn_graph_body(gamma_ref, x_ref, ap_ref, xo_ref, ao_ref):
    xv = x_ref[...]
    ap = ap_ref[...]
    gamma = gamma_ref[0]

    mean = jnp.mean(xv, axis=2, keepdims=True)
    xm = xv - mean
    var = jnp.sum(xm * xm, axis=2, keepdims=True) * (1.0 / (T - 1))
    sinv = 1.0 / (jnp.sqrt(var) + 1e-06)

    acc = jax.lax.dot_general(
        xm, xm, (((2,), (2,)), ((0,), (0,))),
        preferred_element_type=jnp.float32)
    scale = (sinv * (1.0 / T)) * jnp.swapaxes(sinv, 1, 2)
    A = jnp.maximum(acc * scale, 0.0)

    work = A
    for _ in range(K - 1):
        m = jnp.max(work, axis=2, keepdims=True)
        work = jnp.where(work >= m, -1.0, work)
    thr = jnp.max(work, axis=2, keepdims=True)

    mrow = jnp.where(A >= thr, A, 0.0)
    mcol = jnp.where(A >= jnp.swapaxes(thr, 1, 2), A, 0.0)
    S = 0.5 * (mrow + mcol)

    row = jax.lax.broadcasted_iota(jnp.int32, (B, C, C), 1)
    col = jax.lax.broadcasted_iota(jnp.int32, (B, C, C), 2)
    S = jnp.where(row == col, S + 1.0, S)
    deg = jnp.sum(S, axis=2, keepdims=True) + 1e-06
    S = S / deg
    A_out = ALPHA * ap + (1.0 - ALPHA) * S
    ao_ref[...] = A_out

    z = jax.lax.dot_general(
        A_out, xv, (((2,), (1,)), ((0,), (0,))),
        preferred_element_type=jnp.float32)
    xo_ref[...] = xv + gamma * z


def kernel(x, A_prev, gamma):
    gamma_arr = jnp.reshape(gamma.astype(jnp.float32), (1,))
    grid_spec = pltpu.PrefetchScalarGridSpec(
        num_scalar_prefetch=1,
        grid=(N // B,),
        in_specs=[
            pl.BlockSpec((B, C, T), lambda i, g: (i, 0, 0)),
            pl.BlockSpec((B, C, C), lambda i, g: (i, 0, 0)),
        ],
        out_specs=[
            pl.BlockSpec((B, C, T), lambda i, g: (i, 0, 0)),
            pl.BlockSpec((B, C, C), lambda i, g: (i, 0, 0)),
        ],
    )
    x_out, A_out = pl.pallas_call(
        _dyn_graph_body,
        grid_spec=grid_spec,
        out_shape=[
            jax.ShapeDtypeStruct((N, C, T), jnp.float32),
            jax.ShapeDtypeStruct((N, C, C), jnp.float32),
        ],
        compiler_params=pltpu.CompilerParams(
            dimension_semantics=("parallel",),
        ),
    )(gamma_arr, x, A_prev)
    return (x_out, A_out)


# speedup vs baseline: 1.0251x; 1.0251x over previous
import jax
import jax.numpy as jnp
from jax.experimental import pallas as pl
from jax.experimental.pallas import tpu as pltpu

N, C, T = 64, 256, 512
K = 8
ALPHA = 0.8
B = 8


def _dyn_graph_body(gamma_ref, x_ref, ap_ref, eye_ref, xo_ref, ao_ref):
    xv = x_ref[...]
    eye = eye_ref[...]
    gamma = gamma_ref[0]

    acc = jax.lax.dot_general(
        xv, xv, (((2,), (2,)), ((0,), (0,))),
        preferred_element_type=jnp.float32)

    mean = jnp.sum(xv, axis=2, keepdims=True) * (1.0 / T)
    d = jnp.sum(acc * eye, axis=2, keepdims=True)
    var = (d - (mean * mean) * T) * (1.0 / (T - 1))
    sinv = 1.0 / (jnp.sqrt(var) + 1e-06)

    a = sinv * (T ** -0.5)
    q = mean * sinv
    aT = jnp.swapaxes(a, 1, 2)
    qT = jnp.swapaxes(q, 1, 2)
    A = jnp.maximum(acc * (a * aT) - q * qT, 0.0)

    work = A
    for _ in range(K - 1):
        m = jnp.max(work, axis=2, keepdims=True)
        work = jnp.where(work >= m, -1.0, work)
    thr = jnp.max(work, axis=2, keepdims=True)

    mrow = jnp.where(A >= thr, A, 0.0)
    mcol = jnp.where(A >= jnp.swapaxes(thr, 1, 2), A, 0.0)
    S = 0.5 * (mrow + mcol) + eye

    deg = jnp.sum(S, axis=2, keepdims=True) + 1e-06
    rdeg = (1.0 - ALPHA) / deg
    A_out = ALPHA * ap_ref[...] + rdeg * S
    ao_ref[...] = A_out

    z = jax.lax.dot_general(
        gamma * A_out, xv, (((2,), (1,)), ((0,), (0,))),
        preferred_element_type=jnp.float32)
    xo_ref[...] = xv + z


def kernel(x, A_prev, gamma):
    gamma_arr = jnp.reshape(gamma.astype(jnp.float32), (1,))
    eye = jnp.eye(C, dtype=jnp.float32)[None]
    grid_spec = pltpu.PrefetchScalarGridSpec(
        num_scalar_prefetch=1,
        grid=(N // B,),
        in_specs=[
            pl.BlockSpec((B, C, T), lambda i, g: (i, 0, 0)),
            pl.BlockSpec((B, C, C), lambda i, g: (i, 0, 0)),
            pl.BlockSpec((1, C, C), lambda i, g: (0, 0, 0)),
        ],
        out_specs=[
            pl.BlockSpec((B, C, T), lambda i, g: (i, 0, 0)),
            pl.BlockSpec((B, C, C), lambda i, g: (i, 0, 0)),
        ],
    )
    x_out, A_out = pl.pallas_call(
        _dyn_graph_body,
        grid_spec=grid_spec,
        out_shape=[
            jax.ShapeDtypeStruct((N, C, T), jnp.float32),
            jax.ShapeDtypeStruct((N, C, C), jnp.float32),
        ],
        compiler_params=pltpu.CompilerParams(
            dimension_semantics=("parallel",),
        ),
    )(gamma_arr, x, A_prev, eye)
    return (x_out, A_out)
